# Initial kernel scaffold; baseline (speedup 1.0000x reference)
import jax
import jax.numpy as jnp
from jax.experimental import pallas as pl


def kernel(x, edge_index_1, edge_index_2, n_id, W_l1, W_r1, b1, W_l2, W_r2, b2):
    raise NotImplementedError("write your pallas kernel here")


# speedup vs baseline: 7.4759x; 7.4759x over previous
import functools

import jax
import jax.numpy as jnp
from jax import lax
from jax.experimental import pallas as pl
from jax.experimental.pallas import tpu as pltpu
from jax.experimental.pallas import tpu_sc as plsc

N0, N1, N2 = 10000, 2048, 512
D = 128
W = 144
E1, E2 = 320000, 65536
NC, NS = 2, 16
NW = NC * NS
K = 128


def _tc1_body(x_ref, wl_ref, c_ref, wr_ref, b_ref, y_ref, xr_ref):
    x = x_ref[...]
    y_ref[...] = jnp.dot(x, wl_ref[...],
                         preferred_element_type=jnp.float32) + c_ref[...]
    xr_ref[...] = jnp.dot(x, wr_ref[...],
                          preferred_element_type=jnp.float32) + b_ref[...]


def _tc2_body(p_ref, xr_ref, wl_ref, c_ref, wr_ref, b_ref, y_ref, xr2_ref):
    acc = p_ref[0:N1, :] + p_ref[N1:2 * N1, :]
    cnt = jnp.max(acc[:, D:W], axis=1, keepdims=True)
    recip = 1.0 / jnp.maximum(cnt, 1.0)
    h = jnp.maximum(acc[:, :D] * recip + xr_ref[...], 0.0)
    y_ref[...] = jnp.dot(h, wl_ref[...],
                         preferred_element_type=jnp.float32) + c_ref[...]
    xr2_ref[...] = jnp.dot(h[:N2], wr_ref[...],
                           preferred_element_type=jnp.float32) + b_ref[...]


def _tc3_body(p_ref, xr_ref, o_ref):
    acc = p_ref[0:N2, :] + p_ref[N2:2 * N2, :]
    cnt = jnp.max(acc[:, D:W], axis=1, keepdims=True)
    recip = 1.0 / jnp.maximum(cnt, 1.0)
    z = acc[:, :D] * recip + xr_ref[...]
    m = jnp.max(z, axis=1, keepdims=True)
    lse = jnp.log(jnp.sum(jnp.exp(z - m), axis=1, keepdims=True))
    o_ref[...] = z - m - lse


def _make_sc_seg(E, n_dst):
    CH = E // K
    trips = -(-CH // NW)
    rows_per = n_dst // NS
    mesh = plsc.VectorSubcoreMesh(core_axis_name="c", subcore_axis_name="s")

    @functools.partial(
        pl.kernel,
        out_type=jax.ShapeDtypeStruct((NC * n_dst, W), jnp.float32),
        mesh=mesh,
        scratch_types=[
            pltpu.VMEM((K,), jnp.int32),
            pltpu.VMEM((K,), jnp.int32),
            pltpu.VMEM((K, W), jnp.float32),
            pltpu.VMEM_SHARED((n_dst, W), jnp.float32),
            pltpu.SemaphoreType.DMA,
        ],
        compiler_params=pltpu.CompilerParams(use_tc_tiling_on_sc=False),
    )
    def sc_fn(edges_hbm, table_hbm, zeros_hbm, out_hbm,
              src_v, dst_v, rows_v, acc_sh, sem):
        c = lax.axis_index("c")
        s = lax.axis_index("s")
        wid = s * NC + c
        pltpu.sync_copy(zeros_hbm.at[pl.ds(0, rows_per)],
                        acc_sh.at[pl.ds(s * rows_per, rows_per)])
        plsc.subcore_barrier()

        def body(i, carry):
            g = wid + i * NW

            @pl.when(g < CH)
            def _():
                base = g * K
                pltpu.sync_copy(edges_hbm.at[0, pl.ds(base, K)], src_v)
                pltpu.sync_copy(edges_hbm.at[1, pl.ds(base, K)], dst_v)
                pltpu.async_copy(table_hbm.at[src_v], rows_v, sem).wait()
                pltpu.sync_copy(rows_v, acc_sh.at[dst_v], add=True)
            return carry

        lax.fori_loop(0, trips, body, 0)
        plsc.subcore_barrier()
        pltpu.sync_copy(acc_sh.at[pl.ds(s * rows_per, rows_per)],
                        out_hbm.at[pl.ds(c * n_dst + s * rows_per, rows_per)])

    return sc_fn


_sc_seg_1 = _make_sc_seg(E1, N1)
_sc_seg_2 = _make_sc_seg(E2, N2)


def kernel(x, edge_index_1, edge_index_2, n_id, W_l1, W_r1, b1, W_l2, W_r2, b2):
    f32 = jnp.float32
    x2k = x[:N1]
    pad_w = jnp.zeros((D, W - D), f32)
    wl1aug = jnp.concatenate([W_l1, pad_w], axis=1)
    wl2aug = jnp.concatenate([W_l2, pad_w], axis=1)
    caug = jnp.zeros((1, W), f32).at[0, D].set(1.0)
    zeros_blk = jnp.zeros((N1 // NS, W), f32)

    tc1 = pl.pallas_call(_tc1_body, out_shape=[
        jax.ShapeDtypeStruct((N1, W), f32),
        jax.ShapeDtypeStruct((N1, D), f32),
    ])
    y1aug, xr1 = tc1(x2k, wl1aug, caug, W_r1, b1.reshape(1, D))

    p1 = _sc_seg_1(edge_index_1, y1aug, zeros_blk)

    tc2 = pl.pallas_call(_tc2_body, out_shape=[
        jax.ShapeDtypeStruct((N1, W), f32),
        jax.ShapeDtypeStruct((N2, D), f32),
    ])
    y2aug, xr2 = tc2(p1, xr1, wl2aug, caug, W_r2, b2.reshape(1, D))

    p2 = _sc_seg_2(edge_index_2, y2aug, zeros_blk)

    tc3 = pl.pallas_call(_tc3_body, out_shape=jax.ShapeDtypeStruct((N2, D), f32))
    return tc3(p2, xr2)


# speedup vs baseline: 10.7064x; 1.4321x over previous
import functools

import jax
import jax.numpy as jnp
from jax import lax
from jax.experimental import pallas as pl
from jax.experimental.pallas import tpu as pltpu
from jax.experimental.pallas import tpu_sc as plsc

N0, N1, N2 = 10000, 2048, 512
D = 128
W = 144
E1, E2 = 320000, 65536
NC, NS = 2, 16
NW = NC * NS
K = 128


def _tc1_body(x_ref, wl_ref, c_ref, wr_ref, b_ref, y_ref, xr_ref):
    x = x_ref[...]
    y_ref[...] = jnp.dot(x, wl_ref[...],
                         preferred_element_type=jnp.float32) + c_ref[...]
    xr_ref[...] = jnp.dot(x, wr_ref[...],
                          preferred_element_type=jnp.float32) + b_ref[...]


def _tc2_body(p_ref, xr_ref, wl_ref, c_ref, wr_ref, b_ref, y_ref, xr2_ref):
    acc = p_ref[0:N1, :] + p_ref[N1:2 * N1, :]
    cnt = jnp.max(acc[:, D:W], axis=1, keepdims=True)
    recip = 1.0 / jnp.maximum(cnt, 1.0)
    h = jnp.maximum(acc[:, :D] * recip + xr_ref[...], 0.0)
    y_ref[...] = jnp.dot(h, wl_ref[...],
                         preferred_element_type=jnp.float32) + c_ref[...]
    xr2_ref[...] = jnp.dot(h[:N2], wr_ref[...],
                           preferred_element_type=jnp.float32) + b_ref[...]


def _tc3_body(p_ref, xr_ref, o_ref):
    acc = p_ref[0:N2, :] + p_ref[N2:2 * N2, :]
    cnt = jnp.max(acc[:, D:W], axis=1, keepdims=True)
    recip = 1.0 / jnp.maximum(cnt, 1.0)
    z = acc[:, :D] * recip + xr_ref[...]
    m = jnp.max(z, axis=1, keepdims=True)
    lse = jnp.log(jnp.sum(jnp.exp(z - m), axis=1, keepdims=True))
    o_ref[...] = z - m - lse


def _make_sc_seg(E, n_src, n_dst, stage_table=True):
    EP = E // NW
    F = EP // K
    T = EP % K
    HALF = F // 2
    assert F % 2 == 0 and EP * NW == E and T % 8 == 0
    rows_per = n_dst // NS
    src_per = n_src // NS
    mesh = plsc.VectorSubcoreMesh(core_axis_name="c", subcore_axis_name="s")

    scratch = [
        pltpu.VMEM((K,), jnp.int32),
        pltpu.VMEM((K,), jnp.int32),
        pltpu.VMEM((K, W), jnp.float32),
        pltpu.SemaphoreType.DMA,
        pltpu.VMEM((K,), jnp.int32),
        pltpu.VMEM((K,), jnp.int32),
        pltpu.VMEM((K, W), jnp.float32),
        pltpu.SemaphoreType.DMA,
        pltpu.VMEM_SHARED((n_dst, W), jnp.float32),
    ]
    if stage_table:
        scratch += [pltpu.VMEM_SHARED((n_src, W), jnp.float32)]
    if T:
        scratch += [
            pltpu.VMEM((T,), jnp.int32),
            pltpu.VMEM((T,), jnp.int32),
            pltpu.VMEM((T, W), jnp.float32),
        ]

    @functools.partial(
        pl.kernel,
        out_type=jax.ShapeDtypeStruct((NC * n_dst, W), jnp.float32),
        mesh=mesh,
        scratch_types=scratch,
        compiler_params=pltpu.CompilerParams(use_tc_tiling_on_sc=False),
    )
    def sc_fn(edges_hbm, table_hbm, zeros_hbm, out_hbm,
              s0, d0, r0, sem0, s1, d1, r1, sem1, acc_sh, *rest):
        rest = list(rest)
        table = rest.pop(0) if stage_table else table_hbm
        c = lax.axis_index("c")
        s = lax.axis_index("s")
        wid = s * NC + c
        base = wid * EP
        pltpu.sync_copy(zeros_hbm.at[pl.ds(0, rows_per)],
                        acc_sh.at[pl.ds(s * rows_per, rows_per)])
        if stage_table:
            pltpu.sync_copy(table_hbm.at[pl.ds(s * src_per, src_per)],
                            table.at[pl.ds(s * src_per, src_per)])
        plsc.subcore_barrier()

        def fetch(sv, dv, rv, sem, chunk_base, n):
            pltpu.sync_copy(edges_hbm.at[0, pl.ds(chunk_base, n)], sv)
            cp = pltpu.async_copy(table.at[sv], rv, sem)
            pltpu.sync_copy(edges_hbm.at[1, pl.ds(chunk_base, n)], dv)
            return cp

        fetch(s0, d0, r0, sem0, base, K)

        def body(j, carry):
            i1 = 2 * j + 1
            fetch(s1, d1, r1, sem1, base + i1 * K, K)
            pltpu.make_async_copy(table.at[s0], r0, sem0).wait()
            pltpu.sync_copy(r0, acc_sh.at[d0], add=True)

            @pl.when(i1 + 1 < F)
            def _():
                fetch(s0, d0, r0, sem0, base + (i1 + 1) * K, K)

            pltpu.make_async_copy(table.at[s1], r1, sem1).wait()
            pltpu.sync_copy(r1, acc_sh.at[d1], add=True)
            return carry

        lax.fori_loop(0, HALF, body, 0)

        if T:
            st, dt, rt = rest
            fetch(st, dt, rt, sem0, base + F * K, T).wait()
            pltpu.sync_copy(rt, acc_sh.at[dt], add=True)

        plsc.subcore_barrier()
        pltpu.sync_copy(acc_sh.at[pl.ds(s * rows_per, rows_per)],
                        out_hbm.at[pl.ds(c * n_dst + s * rows_per, rows_per)])

    return sc_fn


_sc_seg_1 = _make_sc_seg(E1, N1, N1)
_sc_seg_2 = _make_sc_seg(E2, N1, N2)


def kernel(x, edge_index_1, edge_index_2, n_id, W_l1, W_r1, b1, W_l2, W_r2, b2):
    f32 = jnp.float32
    x2k = x[:N1]
    pad_w = jnp.zeros((D, W - D), f32)
    wl1aug = jnp.concatenate([W_l1, pad_w], axis=1)
    wl2aug = jnp.concatenate([W_l2, pad_w], axis=1)
    caug = jnp.zeros((1, W), f32).at[0, D].set(1.0)
    zeros_blk = jnp.zeros((N1 // NS, W), f32)

    tc1 = pl.pallas_call(_tc1_body, out_shape=[
        jax.ShapeDtypeStruct((N1, W), f32),
        jax.ShapeDtypeStruct((N1, D), f32),
    ])
    y1aug, xr1 = tc1(x2k, wl1aug, caug, W_r1, b1.reshape(1, D))

    p1 = _sc_seg_1(edge_index_1, y1aug, zeros_blk)

    tc2 = pl.pallas_call(_tc2_body, out_shape=[
        jax.ShapeDtypeStruct((N1, W), f32),
        jax.ShapeDtypeStruct((N2, D), f32),
    ])
    y2aug, xr2 = tc2(p1, xr1, wl2aug, caug, W_r2, b2.reshape(1, D))

    p2 = _sc_seg_2(edge_index_2, y2aug, zeros_blk)

    tc3 = pl.pallas_call(_tc3_body, out_shape=jax.ShapeDtypeStruct((N2, D), f32))
    return tc3(p2, xr2)


# speedup vs baseline: 11.3587x; 1.0609x over previous
import functools

import jax
import jax.numpy as jnp
from jax import lax
from jax.experimental import pallas as pl
from jax.experimental.pallas import tpu as pltpu
from jax.experimental.pallas import tpu_sc as plsc

N0, N1, N2 = 10000, 2048, 512
D = 128
W = 144
E1, E2 = 320000, 65536
NC, NS = 2, 16
NW = NC * NS
K = 128


def _tc1_body(x_ref, wl_ref, c_ref, wr_ref, b_ref, y_ref, xr_ref):
    x = x_ref[...]
    y_ref[...] = jnp.dot(x, wl_ref[...],
                         preferred_element_type=jnp.float32) + c_ref[...]
    xr_ref[...] = jnp.dot(x, wr_ref[...],
                          preferred_element_type=jnp.float32) + b_ref[...]


def _tc2_body(p_ref, xr_ref, wl_ref, c_ref, wr_ref, b_ref, y_ref, xr2_ref):
    acc = p_ref[0:N1, :] + p_ref[N1:2 * N1, :]
    cnt = jnp.max(acc[:, D:W], axis=1, keepdims=True)
    recip = 1.0 / jnp.maximum(cnt, 1.0)
    h = jnp.maximum(acc[:, :D] * recip + xr_ref[...], 0.0)
    y_ref[...] = jnp.dot(h, wl_ref[...],
                         preferred_element_type=jnp.float32) + c_ref[...]
    xr2_ref[...] = jnp.dot(h[:N2], wr_ref[...],
                           preferred_element_type=jnp.float32) + b_ref[...]


def _tc3_body(p_ref, xr_ref, o_ref):
    acc = p_ref[0:N2, :] + p_ref[N2:2 * N2, :]
    cnt = jnp.max(acc[:, D:W], axis=1, keepdims=True)
    recip = 1.0 / jnp.maximum(cnt, 1.0)
    z = acc[:, :D] * recip + xr_ref[...]
    m = jnp.max(z, axis=1, keepdims=True)
    lse = jnp.log(jnp.sum(jnp.exp(z - m), axis=1, keepdims=True))
    o_ref[...] = z - m - lse


def _make_sc_seg(E, n_src, n_dst, stage_table=True):
    CH = E // K
    assert CH * K == E
    F = CH // NW
    X = CH - F * NW
    HALF = F // 2
    assert F % 2 == 0
    rows_per = n_dst // NS
    src_per = n_src // NS
    mesh = plsc.VectorSubcoreMesh(core_axis_name="c", subcore_axis_name="s")

    scratch = [
        pltpu.VMEM((F, K), jnp.int32),
        pltpu.VMEM((F, K), jnp.int32),
        pltpu.VMEM((K, W), jnp.float32),
        pltpu.SemaphoreType.DMA,
        pltpu.VMEM((K, W), jnp.float32),
        pltpu.SemaphoreType.DMA,
        pltpu.VMEM_SHARED((n_dst, W), jnp.float32),
    ]
    if stage_table:
        scratch += [pltpu.VMEM_SHARED((n_src, W), jnp.float32)]
    if X:
        scratch += [
            pltpu.VMEM((1, K), jnp.int32),
            pltpu.VMEM((1, K), jnp.int32),
        ]

    @functools.partial(
        pl.kernel,
        out_type=jax.ShapeDtypeStruct((NC * n_dst, W), jnp.float32),
        mesh=mesh,
        scratch_types=scratch,
        compiler_params=pltpu.CompilerParams(use_tc_tiling_on_sc=False),
    )
    def sc_fn(edges_hbm, table_hbm, zeros_hbm, out_hbm,
              src_v, dst_v, r0, sem0, r1, sem1, acc_sh, *rest):
        rest = list(rest)
        table = rest.pop(0) if stage_table else table_hbm
        c = lax.axis_index("c")
        s = lax.axis_index("s")
        wid = s * NC + c
        cbase = wid * F
        pltpu.sync_copy(edges_hbm.at[0, pl.ds(cbase, F), :], src_v)
        pltpu.sync_copy(edges_hbm.at[1, pl.ds(cbase, F), :], dst_v)
        pltpu.sync_copy(zeros_hbm.at[pl.ds(0, rows_per)],
                        acc_sh.at[pl.ds(s * rows_per, rows_per)])
        if stage_table:
            pltpu.sync_copy(table_hbm.at[pl.ds(s * src_per, src_per)],
                            table.at[pl.ds(s * src_per, src_per)])
        plsc.subcore_barrier()

        pltpu.async_copy(table.at[src_v.at[0]], r0, sem0)

        def body(j, carry):
            i0 = 2 * j
            pltpu.async_copy(table.at[src_v.at[i0 + 1]], r1, sem1)
            pltpu.make_async_copy(table.at[src_v.at[i0]], r0, sem0).wait()
            pltpu.sync_copy(r0, acc_sh.at[dst_v.at[i0]], add=True)

            @pl.when(i0 + 2 < F)
            def _():
                pltpu.async_copy(table.at[src_v.at[i0 + 2]], r0, sem0)

            pltpu.make_async_copy(table.at[src_v.at[i0 + 1]], r1, sem1).wait()
            pltpu.sync_copy(r1, acc_sh.at[dst_v.at[i0 + 1]], add=True)
            return carry

        lax.fori_loop(0, HALF, body, 0)

        if X:
            sx, dx = rest

            @pl.when(wid < X)
            def _():
                xc = NW * F + wid
                pltpu.sync_copy(edges_hbm.at[0, pl.ds(xc, 1), :], sx)
                pltpu.sync_copy(edges_hbm.at[1, pl.ds(xc, 1), :], dx)
                pltpu.async_copy(table.at[sx.at[0]], r0, sem0).wait()
                pltpu.sync_copy(r0, acc_sh.at[dx.at[0]], add=True)

        plsc.subcore_barrier()
        pltpu.sync_copy(acc_sh.at[pl.ds(s * rows_per, rows_per)],
                        out_hbm.at[pl.ds(c * n_dst + s * rows_per, rows_per)])

    return sc_fn


_sc_seg_1 = _make_sc_seg(E1, N1, N1)
_sc_seg_2 = _make_sc_seg(E2, N1, N2)


def kernel(x, edge_index_1, edge_index_2, n_id, W_l1, W_r1, b1, W_l2, W_r2, b2):
    f32 = jnp.float32
    x2k = x[:N1]
    pad_w = jnp.zeros((D, W - D), f32)
    wl1aug = jnp.concatenate([W_l1, pad_w], axis=1)
    wl2aug = jnp.concatenate([W_l2, pad_w], axis=1)
    caug = jnp.zeros((1, W), f32).at[0, D].set(1.0)
    zeros_blk = jnp.zeros((N1 // NS, W), f32)

    tc1 = pl.pallas_call(_tc1_body, out_shape=[
        jax.ShapeDtypeStruct((N1, W), f32),
        jax.ShapeDtypeStruct((N1, D), f32),
    ])
    y1aug, xr1 = tc1(x2k, wl1aug, caug, W_r1, b1.reshape(1, D))

    p1 = _sc_seg_1(edge_index_1.reshape(2, E1 // K, K), y1aug, zeros_blk)

    tc2 = pl.pallas_call(_tc2_body, out_shape=[
        jax.ShapeDtypeStruct((N1, W), f32),
        jax.ShapeDtypeStruct((N2, D), f32),
    ])
    y2aug, xr2 = tc2(p1, xr1, wl2aug, caug, W_r2, b2.reshape(1, D))

    p2 = _sc_seg_2(edge_index_2.reshape(2, E2 // K, K), y2aug, zeros_blk)

    tc3 = pl.pallas_call(_tc3_body, out_shape=jax.ShapeDtypeStruct((N2, D), f32))
    return tc3(p2, xr2)


# speedup vs baseline: 13.3432x; 1.1747x over previous
import functools

import jax
import jax.numpy as jnp
from jax import lax
from jax.experimental import pallas as pl
from jax.experimental.pallas import tpu as pltpu
from jax.experimental.pallas import tpu_sc as plsc

N0, N1, N2 = 10000, 2048, 512
D = 128
W = 144
E1, E2 = 320000, 65536
NC, NS = 2, 16
NW = NC * NS
K = 128


def _tc1_body(x_ref, wl_ref, c_ref, wr_ref, b_ref, y_ref, xr_ref):
    x = x_ref[...]
    y_ref[...] = jnp.dot(x, wl_ref[...],
                         preferred_element_type=jnp.float32) + c_ref[...]
    xr_ref[...] = jnp.dot(x, wr_ref[...],
                          preferred_element_type=jnp.float32) + b_ref[...]


def _tc2_body(p_ref, xr_ref, wl_ref, c_ref, wr_ref, b_ref, y_ref, xr2_ref):
    acc = p_ref[0:N1, :] + p_ref[N1:2 * N1, :]
    cnt = jnp.max(acc[:, D:W], axis=1, keepdims=True)
    recip = 1.0 / jnp.maximum(cnt, 1.0)
    h = jnp.maximum(acc[:, :D] * recip + xr_ref[...], 0.0)
    y_ref[...] = jnp.dot(h, wl_ref[...],
                         preferred_element_type=jnp.float32) + c_ref[...]
    xr2_ref[...] = jnp.dot(h[:N2], wr_ref[...],
                           preferred_element_type=jnp.float32) + b_ref[...]


def _tc3_body(p_ref, xr_ref, o_ref):
    acc = p_ref[0:N2, :] + p_ref[N2:2 * N2, :]
    cnt = jnp.max(acc[:, D:W], axis=1, keepdims=True)
    recip = 1.0 / jnp.maximum(cnt, 1.0)
    z = acc[:, :D] * recip + xr_ref[...]
    m = jnp.max(z, axis=1, keepdims=True)
    lse = jnp.log(jnp.sum(jnp.exp(z - m), axis=1, keepdims=True))
    o_ref[...] = z - m - lse


def _make_sc_seg(E, n_src, n_dst, stage_table=True):
    CH = E // K
    assert CH * K == E
    F = CH // NW
    X = CH - F * NW
    HALF = F // 2
    assert F % 2 == 0
    rows_per = n_dst // NS
    src_per = n_src // NS
    mesh = plsc.VectorSubcoreMesh(core_axis_name="c", subcore_axis_name="s")

    scratch = [
        pltpu.VMEM((F, K), jnp.int32),
        pltpu.VMEM((F, K), jnp.int32),
        pltpu.VMEM((K, W), jnp.float32),
        pltpu.SemaphoreType.DMA,
        pltpu.VMEM((K, W), jnp.float32),
        pltpu.SemaphoreType.DMA,
        pltpu.VMEM_SHARED((n_dst, W), jnp.float32),
    ]
    if stage_table:
        scratch += [pltpu.VMEM_SHARED((n_src, W), jnp.float32)]
    if X:
        scratch += [
            pltpu.VMEM((1, K), jnp.int32),
            pltpu.VMEM((1, K), jnp.int32),
        ]

    @functools.partial(
        pl.kernel,
        out_type=jax.ShapeDtypeStruct((NC * n_dst, W), jnp.float32),
        mesh=mesh,
        scratch_types=scratch,
        compiler_params=pltpu.CompilerParams(use_tc_tiling_on_sc=False),
    )
    def sc_fn(edges_hbm, table_hbm, zeros_hbm, out_hbm,
              src_v, dst_v, r0, sem0, r1, sem1, acc_sh, *rest):
        rest = list(rest)
        table = rest.pop(0) if stage_table else table_hbm
        c = lax.axis_index("c")
        s = lax.axis_index("s")
        wid = s * NC + c
        cbase = wid * F
        pltpu.sync_copy(edges_hbm.at[0, pl.ds(cbase, F), :], src_v)
        pltpu.sync_copy(edges_hbm.at[1, pl.ds(cbase, F), :], dst_v)
        pltpu.sync_copy(zeros_hbm.at[pl.ds(0, rows_per)],
                        acc_sh.at[pl.ds(s * rows_per, rows_per)])
        if stage_table:
            pltpu.sync_copy(table_hbm.at[pl.ds(s * src_per, src_per)],
                            table.at[pl.ds(s * src_per, src_per)])
        plsc.subcore_barrier()

        pltpu.async_copy(table.at[src_v.at[0]], r0, sem0)

        def body(j, carry):
            i0 = 2 * j
            pltpu.async_copy(table.at[src_v.at[i0 + 1]], r1, sem1)
            pltpu.make_async_copy(table.at[src_v.at[i0]], r0, sem0).wait()
            pltpu.sync_copy(r0, acc_sh.at[dst_v.at[i0]], add=True)

            @pl.when(i0 + 2 < F)
            def _():
                pltpu.async_copy(table.at[src_v.at[i0 + 2]], r0, sem0)

            pltpu.make_async_copy(table.at[src_v.at[i0 + 1]], r1, sem1).wait()
            pltpu.sync_copy(r1, acc_sh.at[dst_v.at[i0 + 1]], add=True)
            return carry

        lax.fori_loop(0, HALF, body, 0)

        if X:
            sx, dx = rest

            @pl.when(wid < X)
            def _():
                xc = NW * F + wid
                pltpu.sync_copy(edges_hbm.at[0, pl.ds(xc, 1), :], sx)
                pltpu.sync_copy(edges_hbm.at[1, pl.ds(xc, 1), :], dx)
                pltpu.async_copy(table.at[sx.at[0]], r0, sem0).wait()
                pltpu.sync_copy(r0, acc_sh.at[dx.at[0]], add=True)

        plsc.subcore_barrier()
        pltpu.sync_copy(acc_sh.at[pl.ds(s * rows_per, rows_per)],
                        out_hbm.at[pl.ds(c * n_dst + s * rows_per, rows_per)])

    return sc_fn


_sc_seg_1 = _make_sc_seg(E1, N1, N1, stage_table=False)
_sc_seg_2 = _make_sc_seg(E2, N1, N2, stage_table=False)


def kernel(x, edge_index_1, edge_index_2, n_id, W_l1, W_r1, b1, W_l2, W_r2, b2):
    f32 = jnp.float32
    x2k = x[:N1]
    pad_w = jnp.zeros((D, W - D), f32)
    wl1aug = jnp.concatenate([W_l1, pad_w], axis=1)
    wl2aug = jnp.concatenate([W_l2, pad_w], axis=1)
    caug = jnp.zeros((1, W), f32).at[0, D].set(1.0)
    zeros_blk = jnp.zeros((N1 // NS, W), f32)

    tc1 = pl.pallas_call(_tc1_body, out_shape=[
        jax.ShapeDtypeStruct((N1, W), f32),
        jax.ShapeDtypeStruct((N1, D), f32),
    ])
    y1aug, xr1 = tc1(x2k, wl1aug, caug, W_r1, b1.reshape(1, D))

    p1 = _sc_seg_1(edge_index_1.reshape(2, E1 // K, K), y1aug, zeros_blk)

    tc2 = pl.pallas_call(_tc2_body, out_shape=[
        jax.ShapeDtypeStruct((N1, W), f32),
        jax.ShapeDtypeStruct((N2, D), f32),
    ])
    y2aug, xr2 = tc2(p1, xr1, wl2aug, caug, W_r2, b2.reshape(1, D))

    p2 = _sc_seg_2(edge_index_2.reshape(2, E2 // K, K), y2aug, zeros_blk)

    tc3 = pl.pallas_call(_tc3_body, out_shape=jax.ShapeDtypeStruct((N2, D), f32))
    return tc3(p2, xr2)
